# Initial kernel scaffold; baseline (speedup 1.0000x reference)
import jax
import jax.numpy as jnp
from jax.experimental import pallas as pl


def kernel(x, proxies, W1, as1, ad1, b1, W2, as2, ad2, b2, Wm1, bm1, Wm2, bm2, Wf, bf):
    raise NotImplementedError("write your pallas kernel here")


# speedup vs baseline: 1444.6515x; 1444.6515x over previous
import jax
import jax.numpy as jnp
from jax.experimental import pallas as pl

P = 64
N = 1024
T = P + N
HD = 8
OC = 64
D = HD * OC

_F32 = jnp.float32


def _iota(shape, dim):
    return jax.lax.broadcasted_iota(jnp.int32, shape, dim)


def _expand_k():
    return jnp.where(_iota((HD, D), 0) == _iota((HD, D), 1) // OC,
                     1.0, 0.0).astype(_F32)


def _blocksum_m():
    return jnp.where(_iota((D, HD), 0) // OC == _iota((D, HD), 1),
                     1.0, 0.0).astype(_F32)


def _sel_k():
    return _blocksum_m()


def _sel_p():
    return jnp.where(_iota((P, D), 0) == _iota((P, D), 1) % OC,
                     1.0, 0.0).astype(_F32)


def _blockmask():
    return jnp.where(_iota((D, D), 0) // OC == _iota((D, D), 1) // OC,
                     1.0, 0.0).astype(_F32)


def _mm(a, b):
    return jax.lax.dot_general(a, b, (((1,), (0,)), ((), ())),
                               preferred_element_type=_F32)


def _rep64(a):
    return _mm(a, _expand_k())


def _leaky(v):
    return jnp.where(v >= 0, v, 0.2 * v)


def _gat_layer(feats, W, asd, bias, need_prox):
    h = _mm(feats, W)
    scores = _mm(h, asd)
    a_s, a_d = scores[:, :HD], scores[:, HD:]
    h_prox, h_nodes = h[:P], h[P:]
    as_prox, as_nodes = a_s[:P], a_s[P:]
    ad_prox, ad_nodes = a_d[:P], a_d[P:]

    r1 = _mm(as_prox, _expand_k())
    v = jnp.sum(r1 * _sel_p(), axis=0, keepdims=True)
    e_node = _leaky(v + _rep64(ad_nodes))
    ex_node = jnp.exp(e_node)
    ex_self_n = jnp.exp(_leaky(as_nodes + ad_nodes))
    denom_n = _mm(ex_node, _blocksum_m()) + ex_self_n + 1e-16
    alpha_n = ex_node / _rep64(denom_n)
    alpha_self_n = ex_self_n / denom_n
    h_bd = _mm(_sel_p_t(), h_prox) * _blockmask()
    out_nodes = (_mm(alpha_n, h_bd)
                 + _rep64(alpha_self_n) * h_nodes + bias)

    if not need_prox:
        return None, out_nodes

    as_stack = jax.lax.dot_general(_sel_k(), as_nodes,
                                   (((1,), (1,)), ((), ())),
                                   preferred_element_type=_F32)
    y_d = jax.lax.dot_general(_sel_k(), ad_prox, (((1,), (1,)), ((), ())),
                              preferred_element_type=_F32)
    rowsel = jnp.where(_iota((D, P), 0) % OC == _iota((D, P), 1),
                       1.0, 0.0).astype(_F32)
    ad_prox_col = jnp.sum(y_d * rowsel, axis=1, keepdims=True)
    y_s = jax.lax.dot_general(_sel_k(), as_prox, (((1,), (1,)), ((), ())),
                              preferred_element_type=_F32)
    as_prox_col = jnp.sum(y_s * rowsel, axis=1, keepdims=True)

    ex_prox = jnp.exp(_leaky(as_stack + ad_prox_col))
    ex_self_p = jnp.exp(_leaky(as_prox_col + ad_prox_col))
    denom_p = (jnp.sum(ex_prox, axis=1, keepdims=True)
               + ex_self_p + 1e-16)
    alpha_p = ex_prox / denom_p
    r_full = _mm(alpha_p, h_nodes)
    out_prox = _mm(_sel_p(), r_full * _blockmask())
    alpha_self_col = ex_self_p / denom_p
    rep_self = _mm(_sel_p(), alpha_self_col * _blockmask())
    out_prox = out_prox + rep_self * h_prox + bias
    return out_prox, out_nodes


def _sel_p_t():
    return jnp.where(_iota((D, P), 0) % OC == _iota((D, P), 1),
                     1.0, 0.0).astype(_F32)


def _model_body(x_ref, prox_ref, W1_ref, asd1_ref, b1_ref,
                W2_ref, asd2_ref, b2_ref,
                Wm1_ref, bm1_ref, Wm2_ref, bm2_ref, Wf_ref, bf_ref,
                preds_ref, feats_ref):
    feats = jnp.concatenate([prox_ref[...], x_ref[...]], axis=0)

    p1, n1 = _gat_layer(feats, W1_ref[...], asd1_ref[...], b1_ref[...],
                        need_prox=True)
    f1 = jax.nn.relu(jnp.concatenate([p1, n1], axis=0))

    _, n2 = _gat_layer(f1, W2_ref[...], asd2_ref[...], b2_ref[...],
                       need_prox=False)
    f2 = jax.nn.relu(n2)

    hmid = jax.nn.relu(_mm(f2, Wm1_ref[...]) + bm1_ref[...])
    f3 = jax.nn.relu(_mm(hmid, Wm2_ref[...]) + bm2_ref[...])
    preds = _mm(f3, Wf_ref[...]) + bf_ref[...]
    preds_ref[...] = preds
    feats_ref[...] = f3


def _att_blockdiag(att_s, att_d):
    r = jnp.arange(D)[:, None]
    k = jnp.arange(HD)[None, :]
    sel = (r // OC == k).astype(_F32)
    return jnp.concatenate([att_s.reshape(D, 1) * sel,
                            att_d.reshape(D, 1) * sel], axis=1)


def kernel(x, proxies, W1, as1, ad1, b1, W2, as2, ad2, b2,
           Wm1, bm1, Wm2, bm2, Wf, bf):
    out_shape = (jax.ShapeDtypeStruct((N, D), _F32),
                 jax.ShapeDtypeStruct((N, D), _F32))
    preds, feats = pl.pallas_call(
        _model_body,
        out_shape=out_shape,
    )(x, proxies,
      W1, _att_blockdiag(as1, ad1), b1.reshape(1, D),
      W2, _att_blockdiag(as2, ad2), b2.reshape(1, D),
      Wm1, bm1.reshape(1, 4 * D), Wm2, bm2.reshape(1, D),
      Wf, bf.reshape(1, D))
    return preds, feats
